# Initial kernel scaffold; baseline (speedup 1.0000x reference)
import jax
import jax.numpy as jnp
from jax.experimental import pallas as pl


def kernel(x, edge_index, edge_attr, params):
    raise NotImplementedError("write your pallas kernel here")


# speedup vs baseline: 2.0665x; 2.0665x over previous
import jax
import jax.numpy as jnp
from jax.experimental import pallas as pl

_N = 10000
_D = 256


def _post_body(agg_ref, den_ref, h_ref, bo_ref, g_ref, b_ref, o_ref):
    den = den_ref[...]
    conv = agg_ref[...] / (den + 1e-16) + bo_ref[...]
    t = conv + h_ref[...]
    mu = jnp.mean(t, axis=-1, keepdims=True)
    var = jnp.mean((t - mu) ** 2, axis=-1, keepdims=True)
    y = (t - mu) * jax.lax.rsqrt(var + 1e-5) * g_ref[...] + b_ref[...]
    o_ref[...] = jnp.maximum(y, 0.0)


def _post(agg, den, h, bo, g, b):
    return pl.pallas_call(
        _post_body,
        out_shape=jax.ShapeDtypeStruct((_N, _D), jnp.float32),
    )(agg, den.reshape(_N, 1), h, bo.reshape(1, _D), g.reshape(1, _D),
      b.reshape(1, _D))


def kernel(x, edge_index, edge_attr, params):
    src = edge_index[0]
    dst = edge_index[1]
    h = x
    for k in range(3):
        Wl, bl = params[f"Wl{k}"], params[f"bl{k}"]
        Wr, br = params[f"Wr{k}"], params[f"br{k}"]
        We, att = params[f"We{k}"], params[f"att{k}"]
        xl = h @ Wl + bl
        xr = h @ Wr + br
        e_emb = edge_attr @ We
        m = xl[src] + xr[dst] + e_emb
        m_act = jax.nn.leaky_relu(m, negative_slope=0.2)
        alpha = m_act @ att[0]
        ex = jnp.exp(alpha)
        den = jax.ops.segment_sum(ex, dst, num_segments=_N)
        agg = jax.ops.segment_sum(ex[:, None] * xl[src], dst, num_segments=_N)
        h = _post(agg, den, h, params[f"bo{k}"], params[f"ln_g{k}"],
                  params[f"ln_b{k}"])
    return h


# speedup vs baseline: 5.0707x; 2.4537x over previous
import functools

import jax
import jax.numpy as jnp
from jax import lax
from jax.experimental import pallas as pl
from jax.experimental.pallas import tpu as pltpu
from jax.experimental.pallas import tpu_sc as plsc

_N = 10000
_E = 160000
_D = 256
_CH = 160
_NCH = _E // _CH
_F32 = jnp.float32

_mesh = plsc.VectorSubcoreMesh(core_axis_name="c", subcore_axis_name="s")


def _mm_body(h_ref, wl_ref, bl_ref, wr_ref, br_ref,
             xl_ref, xr_ref, xlo_ref, xhi_ref):
    h = h_ref[...]
    yl = jnp.dot(h, wl_ref[...], preferred_element_type=_F32) + bl_ref[...]
    yr = jnp.dot(h, wr_ref[...], preferred_element_type=_F32) + br_ref[...]
    xl_ref[...] = yl
    xr_ref[...] = yr
    xlo_ref[...] = yl[:, :128]
    xhi_ref[...] = yl[:, 128:]


def _mm(h, Wl, bl, Wr, br):
    BN = 1000
    return pl.pallas_call(
        _mm_body,
        grid=(_N // BN,),
        in_specs=[pl.BlockSpec((BN, _D), lambda i: (i, 0)),
                  pl.BlockSpec((_D, _D), lambda i: (0, 0)),
                  pl.BlockSpec((1, _D), lambda i: (0, 0)),
                  pl.BlockSpec((_D, _D), lambda i: (0, 0)),
                  pl.BlockSpec((1, _D), lambda i: (0, 0))],
        out_specs=[pl.BlockSpec((BN, _D), lambda i: (i, 0)),
                   pl.BlockSpec((BN, _D), lambda i: (i, 0)),
                   pl.BlockSpec((BN, 128), lambda i: (i, 0)),
                   pl.BlockSpec((BN, 128), lambda i: (i, 0))],
        out_shape=[jax.ShapeDtypeStruct((_N, _D), _F32),
                   jax.ShapeDtypeStruct((_N, _D), _F32),
                   jax.ShapeDtypeStruct((_N, 128), _F32),
                   jax.ShapeDtypeStruct((_N, 128), _F32)],
    )(h, Wl, bl.reshape(1, _D), Wr, br.reshape(1, _D))


@functools.partial(
    pl.kernel, mesh=_mesh,
    compiler_params=pltpu.CompilerParams(needs_layout_passes=False),
    out_type=[jax.ShapeDtypeStruct((_E,), _F32),
              jax.ShapeDtypeStruct((2, _N), _F32)],
    scratch_types=[
        pltpu.VMEM((_CH,), jnp.int32),
        pltpu.VMEM((_CH,), jnp.int32),
        pltpu.VMEM((_CH,), _F32),
        pltpu.VMEM((_CH,), _F32),
        pltpu.VMEM((_CH, _D), _F32),
        pltpu.VMEM((_CH, _D), _F32),
        pltpu.VMEM((_CH,), _F32),
        pltpu.VMEM((3, _D), _F32),
        pltpu.VMEM((2000,), _F32),
        pltpu.VMEM_SHARED((_N,), _F32),
        pltpu.SemaphoreType.DMA,
    ])
def _pass1(xl_hbm, xr_hbm, src_hbm, dst_hbm, ea0_hbm, ea1_hbm, wa_hbm,
           ex_hbm, den_hbm,
           src_v, dst_v, ea0_v, ea1_v, xl_rows, xr_rows, ex_v, wa_v, zbuf,
           shared_den, sem):
    cid = lax.axis_index("c")
    sid = lax.axis_index("s")
    wid = sid * 2 + cid

    @pl.when(sid == 0)
    def _zero():
        def zb(i, _):
            zbuf[pl.ds(i * 16, 16)] = jnp.zeros((16,), _F32)
            return 0
        lax.fori_loop(0, 125, zb, 0)
        for j in range(5):
            pltpu.sync_copy(zbuf, shared_den.at[pl.ds(j * 2000, 2000)])

    pltpu.sync_copy(wa_hbm, wa_v)
    plsc.subcore_barrier()

    we0 = [wa_v[0, pl.ds(v * 16, 16)] for v in range(16)]
    we1 = [wa_v[1, pl.ds(v * 16, 16)] for v in range(16)]
    attv = [wa_v[2, pl.ds(v * 16, 16)] for v in range(16)]

    def chunk(t, _):
        c = wid + 32 * t

        @pl.when(c < _NCH)
        def _():
            off = c * _CH
            pltpu.sync_copy(src_hbm.at[pl.ds(off, _CH)], src_v)
            pltpu.sync_copy(dst_hbm.at[pl.ds(off, _CH)], dst_v)
            pltpu.sync_copy(ea0_hbm.at[pl.ds(off, _CH)], ea0_v)
            pltpu.sync_copy(ea1_hbm.at[pl.ds(off, _CH)], ea1_v)
            pltpu.async_copy(xl_hbm.at[src_v], xl_rows, sem).wait()
            pltpu.async_copy(xr_hbm.at[dst_v], xr_rows, sem).wait()

            lane = lax.iota(jnp.int32, 16)

            def group(g, _):
                gb = g * 16
                ea0g = ea0_v[pl.ds(gb, 16)]
                ea1g = ea1_v[pl.ds(gb, 16)]
                alpha_g = jnp.zeros((16,), _F32)
                for j in range(16):
                    e = gb + j
                    ea0 = ea0g[j]
                    ea1 = ea1g[j]
                    acc_l = jnp.zeros((16,), _F32)
                    acc_a = jnp.zeros((16,), _F32)
                    for v in range(16):
                        mv = (xl_rows[e, pl.ds(v * 16, 16)]
                              + xr_rows[e, pl.ds(v * 16, 16)]
                              + (ea0 * we0[v] + ea1 * we1[v]))
                        acc_l = acc_l + mv * attv[v]
                        acc_a = acc_a + jnp.abs(mv) * attv[v]
                    alpha = plsc.cumsum(0.6 * acc_l + 0.4 * acc_a)[15]
                    alpha_g = jnp.where(lane == j, alpha, alpha_g)
                ex_v[pl.ds(gb, 16)] = jnp.exp(alpha_g)
                return 0

            lax.fori_loop(0, _CH // 16, group, 0)
            pltpu.sync_copy(ex_v, ex_hbm.at[pl.ds(off, _CH)])
            pltpu.sync_copy(ex_v, shared_den.at[dst_v], add=True)
        return 0

    lax.fori_loop(0, (_NCH + 31) // 32, chunk, 0)
    plsc.subcore_barrier()

    @pl.when(sid == 0)
    def _out():
        pltpu.sync_copy(shared_den, den_hbm.at[cid])


@functools.partial(
    pl.kernel, mesh=_mesh,
    compiler_params=pltpu.CompilerParams(needs_layout_passes=False),
    out_type=jax.ShapeDtypeStruct((2, _N, 128), _F32),
    scratch_types=[
        pltpu.VMEM((_CH,), jnp.int32),
        pltpu.VMEM((_CH,), jnp.int32),
        pltpu.VMEM((_CH,), _F32),
        pltpu.VMEM((_CH, 128), _F32),
        pltpu.VMEM((125, 128), _F32),
        pltpu.VMEM_SHARED((_N, 128), _F32),
        pltpu.SemaphoreType.DMA,
    ])
def _pass2(xlo_hbm, xhi_hbm, src_hbm, dst_hbm, ex_hbm,
           out_hbm,
           src_v, dst_v, ex_v, rows_v, zbuf, shared_acc, sem):
    cid = lax.axis_index("c")
    sid = lax.axis_index("s")

    def zb(i, _):
        for q in range(8):
            zbuf[i, pl.ds(q * 16, 16)] = jnp.zeros((16,), _F32)
        return 0
    lax.fori_loop(0, 125, zb, 0)
    for j in range(5):
        pltpu.sync_copy(zbuf, shared_acc.at[pl.ds(sid * 625 + j * 125, 125)])
    plsc.subcore_barrier()

    def chunk(t, _):
        c = sid + 16 * t

        @pl.when(c < _NCH)
        def _():
            off = c * _CH
            pltpu.sync_copy(src_hbm.at[pl.ds(off, _CH)], src_v)
            pltpu.sync_copy(dst_hbm.at[pl.ds(off, _CH)], dst_v)
            pltpu.sync_copy(ex_hbm.at[pl.ds(off, _CH)], ex_v)

            @pl.when(cid == 0)
            def _():
                pltpu.async_copy(xlo_hbm.at[src_v], rows_v, sem).wait()

            @pl.when(cid == 1)
            def _():
                pltpu.async_copy(xhi_hbm.at[src_v], rows_v, sem).wait()

            def group(g, _):
                gb = g * 16
                exg = ex_v[pl.ds(gb, 16)]
                for j in range(16):
                    e = gb + j
                    exe = exg[j]
                    for q in range(8):
                        rows_v[e, pl.ds(q * 16, 16)] = (
                            rows_v[e, pl.ds(q * 16, 16)] * exe)
                return 0

            lax.fori_loop(0, _CH // 16, group, 0)
            pltpu.sync_copy(rows_v, shared_acc.at[dst_v], add=True)
        return 0

    lax.fori_loop(0, (_NCH + 15) // 16, chunk, 0)
    plsc.subcore_barrier()

    @pl.when(sid == 0)
    def _out():
        pltpu.sync_copy(shared_acc, out_hbm.at[cid])


def _post_body(agg_ref, den_ref, h_ref, bo_ref, g_ref, b_ref, o_ref):
    den = den_ref[:, 0:1] + den_ref[:, 1:2]
    agg = jnp.concatenate([agg_ref[0], agg_ref[1]], axis=-1)
    conv = agg / (den + 1e-16) + bo_ref[...]
    t = conv + h_ref[...]
    mu = jnp.mean(t, axis=-1, keepdims=True)
    var = jnp.mean((t - mu) ** 2, axis=-1, keepdims=True)
    y = (t - mu) * lax.rsqrt(var + 1e-5) * g_ref[...] + b_ref[...]
    o_ref[...] = jnp.maximum(y, 0.0)


def _post(agg2, den_t, h, bo, g, b):
    BN = 1000
    return pl.pallas_call(
        _post_body,
        grid=(_N // BN,),
        in_specs=[pl.BlockSpec((2, BN, 128), lambda i: (0, i, 0)),
                  pl.BlockSpec((BN, 2), lambda i: (i, 0)),
                  pl.BlockSpec((BN, _D), lambda i: (i, 0)),
                  pl.BlockSpec((1, _D), lambda i: (0, 0)),
                  pl.BlockSpec((1, _D), lambda i: (0, 0)),
                  pl.BlockSpec((1, _D), lambda i: (0, 0))],
        out_specs=pl.BlockSpec((BN, _D), lambda i: (i, 0)),
        out_shape=jax.ShapeDtypeStruct((_N, _D), _F32),
    )(agg2, den_t, h, bo.reshape(1, _D), g.reshape(1, _D), b.reshape(1, _D))


def kernel(x, edge_index, edge_attr, params):
    src = edge_index[0]
    dst = edge_index[1]
    ea0 = edge_attr[:, 0]
    ea1 = edge_attr[:, 1]
    h = x
    for k in range(3):
        wa = jnp.concatenate([params[f"We{k}"], params[f"att{k}"]], axis=0)
        xl, xr, xlo, xhi = _mm(h, params[f"Wl{k}"], params[f"bl{k}"],
                               params[f"Wr{k}"], params[f"br{k}"])
        ex, den2 = _pass1(xl, xr, src, dst, ea0, ea1, wa)
        agg2 = _pass2(xlo, xhi, src, dst, ex)
        h = _post(agg2, den2.T, h, params[f"bo{k}"], params[f"ln_g{k}"],
                  params[f"ln_b{k}"])
    return h
